# baseline (speedup 1.0000x reference)
import functools

import jax
import jax.numpy as jnp
from jax import lax
from jax.experimental import pallas as pl
from jax.experimental.pallas import tpu as pltpu
from jax.experimental.pallas import tpu_sc as plsc

_NUM_CORES = 2
_NUM_SUBCORES = 16
_NW = _NUM_CORES * _NUM_SUBCORES
_LANES = 128
_SUBL = 8
_CHUNK = 128


def _make_gather(total_q: int, per_w: int, n_chunk: int, tail: int):
    mesh = plsc.VectorSubcoreMesh(core_axis_name="c", subcore_axis_name="s")
    n_group = n_chunk // 4

    @functools.partial(
        pl.kernel,
        mesh=mesh,
        out_type=jax.ShapeDtypeStruct((total_q, _LANES), jnp.float32),
        scratch_types=[
            pltpu.VMEM((n_chunk, _CHUNK), jnp.int32),
            pltpu.VMEM((_CHUNK, _LANES), jnp.float32),
            pltpu.VMEM((_CHUNK, _LANES), jnp.float32),
            pltpu.VMEM((_CHUNK, _LANES), jnp.float32),
            pltpu.VMEM((_CHUNK, _LANES), jnp.float32),
            pltpu.SemaphoreType.DMA,
            pltpu.SemaphoreType.DMA,
            pltpu.SemaphoreType.DMA,
            pltpu.SemaphoreType.DMA,
            pltpu.SemaphoreType.DMA,
            pltpu.SemaphoreType.DMA,
            pltpu.SemaphoreType.DMA,
            pltpu.SemaphoreType.DMA,
        ],
    )
    def gather_kernel(table_hbm, pidx_hbm, out_hbm, pidx_v,
                      buf0, buf1, buf2, buf3,
                      g0, g1, g2, g3, o0, o1, o2, o3):
        wid = lax.axis_index("s") * _NUM_CORES + lax.axis_index("c")
        pltpu.sync_copy(pidx_hbm.at[wid], pidx_v)
        base_q = jnp.where(wid == _NW - 1, tail, wid * per_w)

        bufs = (buf0, buf1, buf2, buf3)
        gsems = (g0, g1, g2, g3)
        osems = (o0, o1, o2, o3)

        def gather(c, s):
            return pltpu.make_async_copy(
                table_hbm.at[pidx_v.at[c]], bufs[s], gsems[s])

        def out_write(c, s):
            return pltpu.make_async_copy(
                bufs[s], out_hbm.at[pl.ds(base_q + c * _CHUNK, _CHUNK)],
                osems[s])

        gather(0, 0).start()
        gather(1, 1).start()

        def group(m, carry):
            c0 = m * 4
            for s in range(4):
                c = c0 + s
                gather(c, s).wait()
                out_write(c, s).start()
                ss = (s + 2) % 4
                if s < 2:
                    @pl.when(m >= 1)
                    def _wait_prev(c=c, ss=ss):
                        out_write(c - 2, ss).wait()

                    @pl.when(c + 2 < n_chunk)
                    def _prefetch(c=c, ss=ss):
                        gather(c + 2, ss).start()
                else:
                    @pl.when(m < n_group - 1)
                    def _prefetch2(c=c, ss=ss):
                        out_write(c - 2, ss).wait()
                        gather(c + 2, ss).start()
            return carry

        lax.fori_loop(0, n_group, group, 0)
        out_write(n_chunk - 4, 0).wait()
        out_write(n_chunk - 3, 1).wait()
        out_write(n_chunk - 2, 2).wait()
        out_write(n_chunk - 1, 3).wait()

    return gather_kernel


def kernel(last_hidden_state, offsets, mask):
    del mask
    b, t, d = last_hidden_state.shape
    n = offsets.shape[1]
    r = n - 2
    pieces = d // _LANES
    total_q = b * r * pieces

    table = (last_hidden_state.reshape(b * t // _SUBL, _SUBL, pieces, _LANES)
             .transpose(0, 2, 1, 3)
             .reshape(b * t * pieces, _LANES))

    starts = offsets[:, 1 : n - 1, 0]
    rows = starts + (jnp.arange(b, dtype=jnp.int32) * t)[:, None]
    p = ((rows[:, :, None] // _SUBL) * (_SUBL * pieces)
         + jnp.arange(pieces, dtype=jnp.int32)[None, None, :] * _SUBL
         + rows[:, :, None] % _SUBL)
    pidx = p.transpose(1, 2, 0).reshape(-1)

    per_w = -(-total_q // _NW)
    per_w = -(-per_w // (4 * _CHUNK)) * (4 * _CHUNK)
    n_chunk = per_w // _CHUNK
    tail = total_q - per_w
    assert tail % 8 == 0 and tail >= 0
    pidx = jnp.concatenate(
        [pidx[: (_NW - 1) * per_w], pidx[tail:]]).reshape(_NW, n_chunk, _CHUNK)

    out = _make_gather(total_q, per_w, n_chunk, tail)(table, pidx)
    return (out.reshape(r, pieces, b, _LANES)
            .transpose(2, 0, 1, 3)
            .reshape(b, r, d))


# speedup vs baseline: 1.2914x; 1.2914x over previous
import functools

import jax
import jax.numpy as jnp
from jax import lax
from jax.experimental import pallas as pl
from jax.experimental.pallas import tpu as pltpu
from jax.experimental.pallas import tpu_sc as plsc

_NUM_CORES = 2
_NUM_SUBCORES = 16
_NW = _NUM_CORES * _NUM_SUBCORES
_CHUNK = 8
_LANES = 128
_VREG = 16


def _make_gather(total_q: int, per_w: int, n_chunk: int, b: int, d: int,
                 tail: int):
    mesh = plsc.VectorSubcoreMesh(core_axis_name="c", subcore_axis_name="s")
    pieces = d // _LANES
    qchunk = _CHUNK * pieces
    n_pair = n_chunk // 2

    @functools.partial(
        pl.kernel,
        mesh=mesh,
        out_type=jax.ShapeDtypeStruct((total_q, _LANES), jnp.float32),
        scratch_types=[
            pltpu.VMEM((n_chunk, _CHUNK), jnp.int32),
            pltpu.VMEM((_CHUNK, d), jnp.float32),
            pltpu.VMEM((_CHUNK, d), jnp.float32),
            pltpu.VMEM((qchunk, _LANES), jnp.float32),
            pltpu.VMEM((qchunk, _LANES), jnp.float32),
            pltpu.SemaphoreType.DMA,
            pltpu.SemaphoreType.DMA,
            pltpu.SemaphoreType.DMA,
            pltpu.SemaphoreType.DMA,
        ],
    )
    def gather_kernel(table_hbm, gidx_hbm, out_hbm, gidx_v,
                      rows0, rows1, s0, s1, gsem0, gsem1, osem0, osem1):
        wid = lax.axis_index("s") * _NUM_CORES + lax.axis_index("c")
        pltpu.sync_copy(gidx_hbm.at[wid], gidx_v)
        base_q = jnp.where(wid == _NW - 1, tail, wid * per_w) * pieces

        rows = (rows0, rows1)
        gsems = (gsem0, gsem1)
        svmem = (s0, s1)
        osems = (osem0, osem1)

        def gather(c, buf, sem):
            return pltpu.make_async_copy(table_hbm.at[gidx_v.at[c]], buf, sem)

        def out_write(c, buf, sem):
            return pltpu.make_async_copy(
                buf, out_hbm.at[pl.ds(base_q + c * qchunk, qchunk)], sem)

        def interleave(src, dst):
            for j in range(_CHUNK // b):
                for k in range(pieces):
                    for bi in range(b):
                        for v in range(_LANES // _VREG):
                            dst[(j * pieces + k) * b + bi,
                                pl.ds(v * _VREG, _VREG)] = (
                                src[j * b + bi,
                                    pl.ds(k * _LANES + v * _VREG, _VREG)])

        gather(0, rows0, gsem0).start()
        gather(1, rows1, gsem1).start()

        def pair(m, carry):
            for s in range(2):
                c = m * 2 + s
                gather(c, rows[s], gsems[s]).wait()

                @pl.when(m >= 1)
                def _wait_prev(c=c, s=s):
                    out_write(c - 2, svmem[s], osems[s]).wait()

                interleave(rows[s], svmem[s])
                out_write(c, svmem[s], osems[s]).start()

                @pl.when(m < n_pair - 1)
                def _next_gather(c=c, s=s):
                    gather(c + 2, rows[s], gsems[s]).start()
            return carry

        lax.fori_loop(0, n_pair, pair, 0)
        out_write(n_chunk - 2, s0, osem0).wait()
        out_write(n_chunk - 1, s1, osem1).wait()

    return gather_kernel


def kernel(last_hidden_state, offsets, mask):
    del mask
    b, t, d = last_hidden_state.shape
    n = offsets.shape[1]
    r = n - 2
    total_g = b * r
    pieces = d // _LANES
    total_q = total_g * pieces

    starts = offsets[:, 1 : n - 1, 0]
    src = (starts + (jnp.arange(b, dtype=jnp.int32) * t)[:, None]).T.reshape(-1)

    per_w = -(-total_g // _NW)
    per_w = -(-per_w // (2 * _CHUNK)) * (2 * _CHUNK)
    n_chunk = per_w // _CHUNK
    tail = total_g - per_w
    assert tail % 8 == 0 and tail % b == 0 and tail >= 0
    assert _CHUNK % b == 0 and per_w % b == 0
    gidx = jnp.concatenate(
        [src[: (_NW - 1) * per_w], src[tail:]]).reshape(_NW, n_chunk, _CHUNK)

    table = last_hidden_state.reshape(b * t, d)
    out = _make_gather(total_q, per_w, n_chunk, b, d, tail)(table, gidx)
    return (out.reshape(r, pieces, b, _LANES)
            .transpose(2, 0, 1, 3)
            .reshape(b, r, d))


# speedup vs baseline: 1.3027x; 1.0088x over previous
import functools

import jax
import jax.numpy as jnp
from jax import lax
from jax.experimental import pallas as pl
from jax.experimental.pallas import tpu as pltpu
from jax.experimental.pallas import tpu_sc as plsc

_NUM_CORES = 2
_NUM_SUBCORES = 16
_NW = _NUM_CORES * _NUM_SUBCORES
_CHUNK = 8
_LANES = 128
_VREG = 16


def _make_gather(total_q: int, per_w: int, n_chunk: int, b: int, d: int,
                 tail: int):
    mesh = plsc.VectorSubcoreMesh(core_axis_name="c", subcore_axis_name="s")
    pieces = d // _LANES
    qchunk = _CHUNK * pieces
    n_pair = n_chunk // 2

    @functools.partial(
        pl.kernel,
        mesh=mesh,
        out_type=jax.ShapeDtypeStruct((total_q, _LANES), jnp.float32),
        scratch_types=[
            pltpu.VMEM((n_chunk, _CHUNK), jnp.int32),
            pltpu.VMEM((_CHUNK, d), jnp.float32),
            pltpu.VMEM((_CHUNK, d), jnp.float32),
            pltpu.VMEM((qchunk, _LANES), jnp.float32),
            pltpu.VMEM((qchunk, _LANES), jnp.float32),
            pltpu.SemaphoreType.DMA,
            pltpu.SemaphoreType.DMA,
            pltpu.SemaphoreType.DMA,
            pltpu.SemaphoreType.DMA,
        ],
    )
    def gather_kernel(table_hbm, gidx_hbm, out_hbm, gidx_v,
                      rows0, rows1, s0, s1, gsem0, gsem1, osem0, osem1):
        wid = lax.axis_index("s") * _NUM_CORES + lax.axis_index("c")
        pltpu.sync_copy(gidx_hbm.at[wid], gidx_v)
        base_q = jnp.where(wid == _NW - 1, tail, wid * per_w) * pieces

        rows = (rows0, rows1)
        gsems = (gsem0, gsem1)
        svmem = (s0, s1)
        osems = (osem0, osem1)

        def gather(c, buf, sem):
            return pltpu.make_async_copy(table_hbm.at[gidx_v.at[c]], buf, sem)

        half_q = qchunk // 2
        n_j = _CHUNK // b

        def out_write(c, h, buf, sem):
            return pltpu.make_async_copy(
                buf.at[pl.ds(h * half_q, half_q)],
                out_hbm.at[pl.ds(base_q + c * qchunk + h * half_q, half_q)],
                sem)

        def interleave(src, dst, h):
            for j in range(h * n_j // 2, (h + 1) * n_j // 2):
                for k in range(pieces):
                    for bi in range(b):
                        for v in range(_LANES // _VREG):
                            dst[(j * pieces + k) * b + bi,
                                pl.ds(v * _VREG, _VREG)] = (
                                src[j * b + bi,
                                    pl.ds(k * _LANES + v * _VREG, _VREG)])

        gather(0, rows0, gsem0).start()
        gather(1, rows1, gsem1).start()

        def pair(m, carry):
            for s in range(2):
                c = m * 2 + s
                gather(c, rows[s], gsems[s]).wait()

                @pl.when(m >= 1)
                def _wait_prev(c=c, s=s):
                    out_write(c - 2, 0, svmem[s], osems[s]).wait()
                    out_write(c - 2, 1, svmem[s], osems[s]).wait()

                interleave(rows[s], svmem[s], 0)
                out_write(c, 0, svmem[s], osems[s]).start()
                interleave(rows[s], svmem[s], 1)
                out_write(c, 1, svmem[s], osems[s]).start()

                @pl.when(m < n_pair - 1)
                def _next_gather(c=c, s=s):
                    gather(c + 2, rows[s], gsems[s]).start()
            return carry

        lax.fori_loop(0, n_pair, pair, 0)
        out_write(n_chunk - 2, 0, s0, osem0).wait()
        out_write(n_chunk - 2, 1, s0, osem0).wait()
        out_write(n_chunk - 1, 0, s1, osem1).wait()
        out_write(n_chunk - 1, 1, s1, osem1).wait()

    return gather_kernel


def kernel(last_hidden_state, offsets, mask):
    del mask
    b, t, d = last_hidden_state.shape
    n = offsets.shape[1]
    r = n - 2
    total_g = b * r
    pieces = d // _LANES
    total_q = total_g * pieces

    starts = offsets[:, 1 : n - 1, 0]
    src = (starts + (jnp.arange(b, dtype=jnp.int32) * t)[:, None]).T.reshape(-1)

    per_w = -(-total_g // _NW)
    per_w = -(-per_w // (2 * _CHUNK)) * (2 * _CHUNK)
    n_chunk = per_w // _CHUNK
    tail = total_g - per_w
    assert tail % 8 == 0 and tail % b == 0 and tail >= 0
    assert _CHUNK % b == 0 and per_w % b == 0
    gidx = jnp.concatenate(
        [src[: (_NW - 1) * per_w], src[tail:]]).reshape(_NW, n_chunk, _CHUNK)

    table = last_hidden_state.reshape(b * t, d)
    out = _make_gather(total_q, per_w, n_chunk, b, d, tail)(table, gidx)
    return (out.reshape(r, pieces, b, _LANES)
            .transpose(2, 0, 1, 3)
            .reshape(b, r, d))
